# baseline (speedup 1.0000x reference)
import functools

import jax
import jax.numpy as jnp
from jax import lax
from jax.experimental import pallas as pl
from jax.experimental.pallas import tpu as pltpu
from jax.experimental.pallas import tpu_sc as plsc

_D = 32
_B = 16384
_MARGIN = 1.0
_L = 16
_NW = 32
_BW = _B // _NW
_CH = 128
_NCH = _BW // _CH
_NV = _BW // _L

_mesh = plsc.VectorSubcoreMesh(
    core_axis_name="c", subcore_axis_name="s", num_cores=2, num_subcores=16
)


@functools.partial(
    pl.kernel,
    out_type=jax.ShapeDtypeStruct((_NW, _L), jnp.float32),
    mesh=_mesh,
    scratch_types=(
        [pltpu.VMEM((_NCH, _CH), jnp.int32) for _ in range(6)]
        + [pltpu.VMEM((_BW,), jnp.float32) for _ in range(6)]
        + [pltpu.VMEM((_BW,), jnp.float32) for _ in range(2)]
        + [pltpu.VMEM((_L,), jnp.float32), pltpu.SemaphoreType.DMA]
    ),
    compiler_params=pltpu.CompilerParams(
        needs_layout_passes=False, use_tc_tiling_on_sc=False
    ),
)
def _transe_sc(ent_hbm, rel_hbm,
               ph_hbm, pr_hbm, pt_hbm, nh_hbm, nr_hbm, nt_hbm,
               out_hbm,
               iph, ipr, ipt, inh, inr, int_,
               bph, bpr, bpt, bnh, bnr, bnt,
               accp, accn,
               loss_v, sem):
    wid = lax.axis_index("s") * 2 + lax.axis_index("c")

    idx_refs = (iph, ipr, ipt, inh, inr, int_)
    idx_hbms = (ph_hbm, pr_hbm, pt_hbm, nh_hbm, nr_hbm, nt_hbm)
    bufs = (bph, bpr, bpt, bnh, bnr, bnt)
    tables = (ent_hbm, rel_hbm, ent_hbm, ent_hbm, rel_hbm, ent_hbm)

    copies = [
        pltpu.async_copy(h.at[wid], r, sem) for h, r in zip(idx_hbms, idx_refs)
    ]
    for c in copies:
        c.wait()

    zero = lax.broadcast(jnp.float32(0.0), (_L,))
    for v in range(_NV):
        sl = pl.ds(v * _L, _L)
        accp[sl] = zero
        accn[sl] = zero

    def dim_body(d, carry):
        gathers = []
        for tab, iref, bref in zip(tables, idx_refs, bufs):
            plane = tab.at[d]
            for c in range(_NCH):
                gathers.append(
                    pltpu.async_copy(
                        plane.at[iref.at[c]],
                        bref.at[pl.ds(c * _CH, _CH)],
                        sem,
                    )
                )
        for g in gathers:
            g.wait()
        for v in range(_NV):
            sl = pl.ds(v * _L, _L)
            accp[sl] = accp[sl] + jnp.abs(bph[sl] + bpr[sl] - bpt[sl])
            accn[sl] = accn[sl] + jnp.abs(bnh[sl] + bnr[sl] - bnt[sl])
        return carry

    lax.fori_loop(0, _D, dim_body, jnp.int32(0))

    loss = zero
    for v in range(_NV):
        sl = pl.ds(v * _L, _L)
        loss = loss + jnp.maximum(
            accp[sl] - accn[sl] + jnp.float32(_MARGIN), zero
        )
    loss_v[...] = loss
    pltpu.sync_copy(loss_v, out_hbm.at[wid])


def kernel(positive_triplets, negative_triplets, entity_emb, relation_emb):
    cols = [
        arr[:, c].reshape(_NW, _NCH, _CH)
        for arr in (positive_triplets, negative_triplets)
        for c in range(3)
    ]
    partials = _transe_sc(entity_emb.T, relation_emb.T, *cols)
    return jnp.sum(partials) / jnp.float32(_B)


# speedup vs baseline: 5.4352x; 5.4352x over previous
import functools

import jax
import jax.numpy as jnp
from jax import lax
from jax.experimental import pallas as pl
from jax.experimental.pallas import tpu as pltpu
from jax.experimental.pallas import tpu_sc as plsc

_D = 32
_B = 16384
_MARGIN = 1.0
_L = 16
_NW = 32
_BW = _B // _NW
_CH = 128
_NCH = _BW // _CH
_NVC = _CH // _L

_mesh = plsc.VectorSubcoreMesh(
    core_axis_name="c", subcore_axis_name="s", num_cores=2, num_subcores=16
)


@functools.partial(
    pl.kernel,
    out_type=jax.ShapeDtypeStruct((_NW, _L), jnp.float32),
    mesh=_mesh,
    scratch_types=(
        [pltpu.VMEM((_NCH, _CH), jnp.int32) for _ in range(6)]
        + [pltpu.VMEM((_BW,), jnp.int32) for _ in range(6)]
        + [pltpu.VMEM((_CH, _CH), jnp.float32) for _ in range(6)]
        + [pltpu.VMEM((_L,), jnp.float32), pltpu.SemaphoreType.DMA]
    ),
    compiler_params=pltpu.CompilerParams(
        needs_layout_passes=False, use_tc_tiling_on_sc=True
    ),
)
def _transe_sc(ent_hbm, rel_hbm,
               p0, p1, p2, p3, p4, p5,
               s0, s1, s2, s3, s4, s5,
               out_hbm,
               ip0, ip1, ip2, ip3, ip4, ip5,
               is0, is1, is2, is3, is4, is5,
               b0, b1, b2, b3, b4, b5,
               loss_v, sem):
    wid = lax.axis_index("s") * 2 + lax.axis_index("c")

    p_hbms = (p0, p1, p2, p3, p4, p5)
    s_hbms = (s0, s1, s2, s3, s4, s5)
    ip_refs = (ip0, ip1, ip2, ip3, ip4, ip5)
    is_refs = (is0, is1, is2, is3, is4, is5)
    bufs = (b0, b1, b2, b3, b4, b5)
    tables = (ent_hbm, rel_hbm, ent_hbm, ent_hbm, rel_hbm, ent_hbm)

    copies = [pltpu.async_copy(h.at[wid], r, sem)
              for h, r in zip(p_hbms + s_hbms, ip_refs + is_refs)]
    for c in copies:
        c.wait()

    lane = lax.iota(jnp.int32, _L)
    zero = lax.broadcast(jnp.float32(0.0), (_L,))
    loss = zero

    for c in range(_NCH):
        gathers = [
            pltpu.async_copy(tab.at[iref.at[c]], bref, sem)
            for tab, iref, bref in zip(tables, ip_refs, bufs)
        ]
        for g in gathers:
            g.wait()

        def vreg_body(v, loss_sum, _c=c):
            row = lane + v * _L
            off = _c * _CH
            cols = [plsc.load_gather(sref, [row + off]) for sref in is_refs]
            acc_p = zero
            acc_n = zero
            for d in range(_D):
                hp = plsc.load_gather(b0, [row, cols[0] + d])
                rp = plsc.load_gather(b1, [row, cols[1] + d])
                tp = plsc.load_gather(b2, [row, cols[2] + d])
                acc_p = acc_p + jnp.abs(hp + rp - tp)
                hn = plsc.load_gather(b3, [row, cols[3] + d])
                rn = plsc.load_gather(b4, [row, cols[4] + d])
                tn = plsc.load_gather(b5, [row, cols[5] + d])
                acc_n = acc_n + jnp.abs(hn + rn - tn)
            hinge = jnp.maximum(acc_p - acc_n + jnp.float32(_MARGIN), zero)
            return loss_sum + hinge

        loss = lax.fori_loop(0, _NVC, vreg_body, loss)

    loss_v[...] = loss
    pltpu.sync_copy(loss_v, out_hbm.at[wid])


def kernel(positive_triplets, negative_triplets, entity_emb, relation_emb):
    packed = []
    offs = []
    for arr in (positive_triplets, negative_triplets):
        for c in range(3):
            col = arr[:, c]
            packed.append((col >> 2).reshape(_NW, _NCH, _CH))
            offs.append(((col & 3) * _D).reshape(_NW, _BW))
    partials = _transe_sc(
        entity_emb.reshape(250000, 128),
        relation_emb.reshape(250000, 128),
        *packed, *offs,
    )
    return jnp.sum(partials) / jnp.float32(_B)
